# Initial kernel scaffold; baseline (speedup 1.0000x reference)
import jax
import jax.numpy as jnp
from jax.experimental import pallas as pl


def kernel(x, med, upp, steps, beta):
    raise NotImplementedError("write your pallas kernel here")


# speedup vs baseline: 11.0009x; 11.0009x over previous
import functools

import jax
import jax.numpy as jnp
import numpy as np
from jax import lax
from jax.experimental import pallas as pl
from jax.experimental.pallas import tpu as pltpu

EPS_ = 1e-07
H_ = 2048
N_ = 16384
Q_ = 0.841
BC_ = 256

J_MED = N_ // 2 - 1
AQ_ = np.float32(Q_) * np.float32(N_ - 1)
J_UPP = int(np.floor(AQ_))
FRAC_ = float(AQ_ - np.float32(J_UPP))

MIN32 = np.int32(-2147483648)
MAX32 = np.int32(2147483647)


def _to_key(b):
    m = lax.shift_right_arithmetic(b, 31)
    return lax.bitwise_xor(b, lax.shift_right_logical(m, 1))


def _select_rank(keys, j):
    ukey_prefix = jnp.zeros((1, BC_), jnp.int32)

    def body(i, p):
        bit = lax.shift_left(jnp.int32(1), (31 - i).astype(jnp.int32))
        cand_u = lax.bitwise_or(p, bit)
        cand_s = cand_u ^ MIN32
        cnt = jnp.sum((keys < cand_s).astype(jnp.int32), axis=0,
                      keepdims=True)
        return jnp.where(cnt <= j, cand_u, p)

    p = lax.fori_loop(0, 32, body, ukey_prefix)
    return p ^ MIN32


def _pair(keys, k_lo, j):
    gt = keys > k_lo
    cnt_gt = jnp.sum(gt.astype(jnp.int32), axis=0, keepdims=True)
    succ = jnp.min(jnp.where(gt, keys, MAX32), axis=0, keepdims=True)
    count_le = N_ - cnt_gt
    k_hi = jnp.where(count_le >= j + 2, k_lo, succ)
    return k_lo, k_hi


def _key_to_f32(k):
    m = lax.shift_right_arithmetic(k, 31)
    b = lax.bitwise_xor(k, lax.shift_right_logical(m, 1))
    return lax.bitcast_convert_type(b, jnp.float32)


def _body(x_ref, med_ref, upp_ref, sc_ref, om_ref, os_ref, keys_ref):
    xb = x_ref[...]
    keys_ref[...] = _to_key(lax.bitcast_convert_type(xb, jnp.int32))
    keys = keys_ref[...]

    km0 = _select_rank(keys, J_MED)
    ku0 = _select_rank(keys, J_UPP)
    km0, km1 = _pair(keys, km0, J_MED)
    ku0, ku1 = _pair(keys, ku0, J_UPP)

    vm0 = _key_to_f32(km0)
    vm1 = _key_to_f32(km1)
    vu0 = _key_to_f32(ku0)
    vu1 = _key_to_f32(ku1)

    new_med = 0.5 * (vm0 + vm1)
    new_upp = vu0 * (1.0 - FRAC_) + vu1 * FRAC_

    bpow = sc_ref[0]
    div = sc_ref[1]
    trig = sc_ref[2]

    med = med_ref[...].reshape(1, BC_)
    upp = upp_ref[...].reshape(1, BC_)
    med_u = bpow * med + (1.0 - bpow) * new_med
    upp_u = bpow * upp + (1.0 - bpow) * new_upp
    med_f = trig * med + (1.0 - trig) * med_u
    upp_f = trig * upp + (1.0 - trig) * upp_u
    adj_med = med_f / (div + EPS_)
    adj_upp = upp_f / (div + EPS_)
    om_ref[...] = adj_med.reshape(BC_)
    os_ref[...] = (adj_upp - adj_med + EPS_).reshape(BC_)


@jax.jit
def _run(xr, med, upp, scalars):
    grid = H_ // BC_
    return pl.pallas_call(
        _body,
        grid=(grid,),
        in_specs=[
            pl.BlockSpec((N_, BC_), lambda i: (0, i)),
            pl.BlockSpec((BC_,), lambda i: (i,)),
            pl.BlockSpec((BC_,), lambda i: (i,)),
            pl.BlockSpec(memory_space=pltpu.SMEM),
        ],
        out_specs=[
            pl.BlockSpec((BC_,), lambda i: (i,)),
            pl.BlockSpec((BC_,), lambda i: (i,)),
        ],
        out_shape=[
            jax.ShapeDtypeStruct((H_,), jnp.float32),
            jax.ShapeDtypeStruct((H_,), jnp.float32),
        ],
        scratch_shapes=[pltpu.VMEM((N_, BC_), jnp.int32)],
    )(xr, med, upp, scalars)


def kernel(x, med, upp, steps, beta):
    xr = x[:4].reshape(N_, H_).astype(jnp.float32)
    delta = 1.0
    bpow = beta ** delta
    trig = (steps > 1.0).astype(jnp.float32)
    new_steps = steps + delta
    steps_f = jnp.where(steps > 1.0, steps, new_steps)
    div = 1.0 - beta ** steps_f
    scalars = jnp.stack([bpow, div, trig])
    return _run(xr, med, upp, scalars)
